# baseline (speedup 1.0000x reference)
import functools

import jax
import jax.numpy as jnp
from jax.experimental import pallas as pl
from jax.experimental.pallas import tpu as pltpu

_TM = 400
_NBUF = 3


def _gcn_kernel(nblocks, adj_any, x_ref, w_ref, b_ref, out_ref, buf_ref, sem):
    for k in range(_NBUF):
        pltpu.make_async_copy(
            adj_any.at[pl.ds(k * _TM, _TM), :], buf_ref.at[k], sem.at[k]
        ).start()
    supp = jnp.dot(x_ref[...], w_ref[...], preferred_element_type=jnp.float32)
    bias_row = b_ref[...]

    def body(i, supp):
        slot = jax.lax.rem(i, _NBUF)
        pltpu.make_async_copy(
            adj_any.at[pl.ds(i * _TM, _TM), :], buf_ref.at[slot], sem.at[slot]
        ).wait()
        acc = jnp.dot(buf_ref[slot], supp, preferred_element_type=jnp.float32)
        out_ref[pl.ds(i * _TM, _TM), :] = jax.nn.sigmoid(acc + bias_row)

        @pl.when(i + _NBUF < nblocks)
        def _refill():
            pltpu.make_async_copy(
                adj_any.at[pl.ds((i + _NBUF) * _TM, _TM), :],
                buf_ref.at[slot],
                sem.at[slot],
            ).start()

        return supp

    jax.lax.fori_loop(0, nblocks, body, supp)


def kernel(input, adj, weight, bias):
    n, in_f = input.shape
    out_f = weight.shape[1]
    bias2d = bias.reshape(1, out_f)
    nblocks = n // _TM
    return pl.pallas_call(
        functools.partial(_gcn_kernel, nblocks),
        in_specs=[
            pl.BlockSpec(memory_space=pltpu.MemorySpace.HBM),
            pl.BlockSpec(memory_space=pltpu.MemorySpace.VMEM),
            pl.BlockSpec(memory_space=pltpu.MemorySpace.VMEM),
            pl.BlockSpec(memory_space=pltpu.MemorySpace.VMEM),
        ],
        out_specs=pl.BlockSpec(memory_space=pltpu.MemorySpace.VMEM),
        out_shape=jax.ShapeDtypeStruct((n, out_f), jnp.float32),
        scratch_shapes=[
            pltpu.VMEM((_NBUF, _TM, n), jnp.float32),
            pltpu.SemaphoreType.DMA((_NBUF,)),
        ],
        compiler_params=pltpu.CompilerParams(
            vmem_limit_bytes=63 * 1024 * 1024,
        ),
    )(adj, input, weight, bias2d)


# speedup vs baseline: 1.0238x; 1.0238x over previous
import jax
import jax.numpy as jnp
from jax.experimental import pallas as pl
from jax.experimental.pallas import tpu as pltpu

_TM = 416


def _gcn_block_kernel(adj_ref, x_ref, w_ref, b_ref, out_ref, supp_ref):
    @pl.when(pl.program_id(0) == 0)
    def _compute_support():
        supp_ref[...] = jnp.dot(
            x_ref[...], w_ref[...], preferred_element_type=jnp.float32
        )

    acc = jnp.dot(adj_ref[...], supp_ref[...], preferred_element_type=jnp.float32)
    out_ref[...] = jax.nn.sigmoid(acc + b_ref[...])


def kernel(input, adj, weight, bias):
    n, in_f = input.shape
    out_f = weight.shape[1]
    bias2d = bias.reshape(1, out_f)
    grid = (pl.cdiv(n, _TM),)
    return pl.pallas_call(
        _gcn_block_kernel,
        grid=grid,
        in_specs=[
            pl.BlockSpec((_TM, n), lambda i: (24 - i, 0)),
            pl.BlockSpec((n, in_f), lambda i: (0, 0)),
            pl.BlockSpec((in_f, out_f), lambda i: (0, 0)),
            pl.BlockSpec((1, out_f), lambda i: (0, 0)),
        ],
        out_specs=pl.BlockSpec((_TM, out_f), lambda i: (24 - i, 0)),
        out_shape=jax.ShapeDtypeStruct((n, out_f), jnp.float32),
        scratch_shapes=[pltpu.VMEM((n, out_f), jnp.float32)],
        compiler_params=pltpu.CompilerParams(
            dimension_semantics=("arbitrary",),
        ),
    )(adj, input, weight, bias2d)


# speedup vs baseline: 1.0318x; 1.0079x over previous
import jax
import jax.numpy as jnp
from jax.experimental import pallas as pl
from jax.experimental.pallas import tpu as pltpu

_TM = 416


def _gcn_block_kernel(adj_ref, x_ref, w_ref, b_ref, out_ref, supp_ref):
    @pl.when(pl.program_id(0) == 0)
    def _compute_support():
        supp_ref[...] = jnp.dot(
            x_ref[...], w_ref[...], preferred_element_type=jnp.float32
        )

    acc = jnp.dot(adj_ref[...], supp_ref[...], preferred_element_type=jnp.float32)
    out_ref[...] = jax.nn.sigmoid(acc + b_ref[...])


def kernel(input, adj, weight, bias):
    n, in_f = input.shape
    out_f = weight.shape[1]
    bias2d = bias.reshape(1, out_f)
    grid = (pl.cdiv(n, _TM),)
    return pl.pallas_call(
        _gcn_block_kernel,
        grid=grid,
        in_specs=[
            pl.BlockSpec((_TM, n), lambda i: ((i + 24) % 25, 0)),
            pl.BlockSpec((n, in_f), lambda i: (0, 0)),
            pl.BlockSpec((in_f, out_f), lambda i: (0, 0)),
            pl.BlockSpec((1, out_f), lambda i: (0, 0)),
        ],
        out_specs=pl.BlockSpec((_TM, out_f), lambda i: ((i + 24) % 25, 0)),
        out_shape=jax.ShapeDtypeStruct((n, out_f), jnp.float32),
        scratch_shapes=[pltpu.VMEM((n, out_f), jnp.float32)],
        compiler_params=pltpu.CompilerParams(
            dimension_semantics=("arbitrary",),
        ),
    )(adj, input, weight, bias2d)


# speedup vs baseline: 1.0485x; 1.0161x over previous
import jax
import jax.numpy as jnp
from jax.experimental import pallas as pl
from jax.experimental.pallas import tpu as pltpu

_TM = 400


def _gcn_block_kernel(adj_ref, x_ref, w_ref, b_ref, out_ref, supp_ref):
    @pl.when(pl.program_id(0) == 0)
    def _compute_support():
        supp_ref[...] = jnp.dot(
            x_ref[...], w_ref[...], preferred_element_type=jnp.float32
        )

    acc = jnp.dot(adj_ref[...], supp_ref[...], preferred_element_type=jnp.float32)
    out_ref[...] = jax.nn.sigmoid(acc + b_ref[...])


def kernel(input, adj, weight, bias):
    n, in_f = input.shape
    out_f = weight.shape[1]
    bias2d = bias.reshape(1, out_f)
    grid = (n // _TM,)
    return pl.pallas_call(
        _gcn_block_kernel,
        grid=grid,
        in_specs=[
            pl.BlockSpec((_TM, n), lambda i: (i, 0)),
            pl.BlockSpec((n, in_f), lambda i: (0, 0)),
            pl.BlockSpec((in_f, out_f), lambda i: (0, 0)),
            pl.BlockSpec((1, out_f), lambda i: (0, 0)),
        ],
        out_specs=pl.BlockSpec((_TM, out_f), lambda i: (i, 0)),
        out_shape=jax.ShapeDtypeStruct((n, out_f), jnp.float32),
        scratch_shapes=[pltpu.VMEM((n, out_f), jnp.float32)],
        compiler_params=pltpu.CompilerParams(
            dimension_semantics=("arbitrary",),
        ),
    )(adj, input, weight, bias2d)
